# baseline (speedup 1.0000x reference)
import functools

import jax
import jax.numpy as jnp
from jax import lax
from jax.experimental import pallas as pl
from jax.experimental.pallas import tpu as pltpu
from jax.experimental.pallas import tpu_sc as plsc

_INPUT_DIM = 512
_LATENT_DIM = 64
_NUM_EMBED = 1024
_BATCH = 16384

_BM = 1024
_BM_DEC = 2048
_N_BLOCKS = _BATCH // _BM


def _argmin_body(x_ref, we_ref, be_ref, e_ref, idx_ref):
    z = jnp.dot(x_ref[...], we_ref[...], preferred_element_type=jnp.float32)
    z = jnp.maximum(z + be_ref[...], 0.0)
    sim = jnp.dot(z, e_ref[...], preferred_element_type=jnp.float32)
    zsq = jnp.sum(z * z, axis=1, keepdims=True)
    esq = jnp.sum(e_ref[...] * e_ref[...], axis=0, keepdims=True)
    dist = zsq + esq - 2.0 * sim
    idx_ref[0, 0, :] = jnp.argmin(dist, axis=1).astype(jnp.int32)


def _compute_indices(x, W_enc, b_enc, embeddings):
    out = pl.pallas_call(
        _argmin_body,
        grid=(_N_BLOCKS,),
        in_specs=[
            pl.BlockSpec((_BM, _INPUT_DIM), lambda i: (i, 0)),
            pl.BlockSpec((_INPUT_DIM, _LATENT_DIM), lambda i: (0, 0)),
            pl.BlockSpec((1, _LATENT_DIM), lambda i: (0, 0)),
            pl.BlockSpec((_LATENT_DIM, _NUM_EMBED), lambda i: (0, 0)),
        ],
        out_specs=pl.BlockSpec((1, 1, _BM), lambda i: (i, 0, 0)),
        out_shape=jax.ShapeDtypeStruct((_N_BLOCKS, 1, _BM), jnp.int32),
    )(x, W_enc, b_enc.reshape(1, _LATENT_DIM), embeddings)
    return out.reshape(_BATCH)


def _transpose_body(e_ref, t_ref):
    t_ref[...] = jnp.concatenate(
        [e_ref[...].T, jnp.zeros((_NUM_EMBED, _PAD - _LATENT_DIM), jnp.float32)],
        axis=1,
    )


_PAD = 128


def _compute_codebook_rows(embeddings):
    return pl.pallas_call(
        _transpose_body,
        out_shape=jax.ShapeDtypeStruct((_NUM_EMBED, _PAD), jnp.float32),
    )(embeddings)


def _make_gather():
    info = plsc.get_sparse_core_info()
    nc, ns = info.num_cores, info.num_subcores
    nw = nc * ns
    b_per_w = _BATCH // nw
    mesh = plsc.VectorSubcoreMesh(core_axis_name="c", subcore_axis_name="s")

    @functools.partial(
        pl.kernel, mesh=mesh,
        out_type=jax.ShapeDtypeStruct((_BATCH, _PAD), jnp.float32),
        scratch_types=[
            pltpu.VMEM((b_per_w,), jnp.int32),
            pltpu.VMEM((b_per_w, _PAD), jnp.float32),
            pltpu.SemaphoreType.DMA,
        ],
    )
    def gather(table_hbm, idx_hbm, out_hbm, idx_v, rows_v, sem):
        wid = lax.axis_index("s") * nc + lax.axis_index("c")
        base = wid * b_per_w
        pltpu.sync_copy(idx_hbm.at[pl.ds(base, b_per_w)], idx_v)
        pltpu.async_copy(table_hbm.at[idx_v], rows_v, sem).wait()
        pltpu.sync_copy(rows_v, out_hbm.at[pl.ds(base, b_per_w)])

    return gather


def _decode_body(q_ref, wd_ref, bd_ref, out_ref):
    q = q_ref[...][:, :_LATENT_DIM]
    out_ref[...] = (
        jnp.dot(q, wd_ref[...], preferred_element_type=jnp.float32)
        + bd_ref[...]
    )


def _decode(quantized, W_dec, b_dec):
    return pl.pallas_call(
        _decode_body,
        grid=(_BATCH // _BM_DEC,),
        in_specs=[
            pl.BlockSpec((_BM_DEC, _PAD), lambda i: (i, 0)),
            pl.BlockSpec((_LATENT_DIM, _INPUT_DIM), lambda i: (0, 0)),
            pl.BlockSpec((1, _INPUT_DIM), lambda i: (0, 0)),
        ],
        out_specs=pl.BlockSpec((_BM_DEC, _INPUT_DIM), lambda i: (i, 0)),
        out_shape=jax.ShapeDtypeStruct((_BATCH, _INPUT_DIM), jnp.float32),
    )(quantized, W_dec, b_dec.reshape(1, _INPUT_DIM))


def kernel(x, W_enc, b_enc, W_dec, b_dec, embeddings):
    idx = _compute_indices(x, W_enc, b_enc, embeddings)
    table = _compute_codebook_rows(embeddings)
    quantized = _make_gather()(table, idx)
    return _decode(quantized, W_dec, b_dec)


# speedup vs baseline: 1.1276x; 1.1276x over previous
import functools

import jax
import jax.numpy as jnp
from jax import lax
from jax.experimental import pallas as pl
from jax.experimental.pallas import tpu as pltpu
from jax.experimental.pallas import tpu_sc as plsc

_INPUT_DIM = 512
_LATENT_DIM = 64
_NUM_EMBED = 1024
_BATCH = 16384

_BM = 1024
_PAD = 128
_BM_DEC = 2048
_N_BLOCKS = _BATCH // _BM


def _argmin_body(x_ref, we_ref, be_ref, e_ref, idx_ref):
    z = jnp.dot(x_ref[...], we_ref[...], preferred_element_type=jnp.float32)
    z = jnp.maximum(z + be_ref[...], 0.0)
    sim = jnp.dot(z, e_ref[...], preferred_element_type=jnp.float32)
    zsq = jnp.sum(z * z, axis=1, keepdims=True)
    esq = jnp.sum(e_ref[...] * e_ref[...], axis=0, keepdims=True)
    dist = zsq + esq - 2.0 * sim
    idx_ref[0, 0, :] = jnp.argmin(dist, axis=1).astype(jnp.int32)


def _compute_indices(x, W_enc, b_enc, embeddings):
    out = pl.pallas_call(
        _argmin_body,
        grid=(_N_BLOCKS,),
        in_specs=[
            pl.BlockSpec((_BM, _INPUT_DIM), lambda i: (i, 0)),
            pl.BlockSpec((_INPUT_DIM, _LATENT_DIM), lambda i: (0, 0)),
            pl.BlockSpec((1, _LATENT_DIM), lambda i: (0, 0)),
            pl.BlockSpec((_LATENT_DIM, _NUM_EMBED), lambda i: (0, 0)),
        ],
        out_specs=pl.BlockSpec((1, 1, _BM), lambda i: (i, 0, 0)),
        out_shape=jax.ShapeDtypeStruct((_N_BLOCKS, 1, _BM), jnp.int32),
    )(x, W_enc, b_enc.reshape(1, _LATENT_DIM), embeddings)
    return out.reshape(_BATCH)


def _make_gather():
    info = plsc.get_sparse_core_info()
    nc, ns = info.num_cores, info.num_subcores
    nw = nc * ns
    b_per_w = _BATCH // nw
    qrows = b_per_w // 4
    n_grp = qrows // 16
    mesh = plsc.VectorSubcoreMesh(core_axis_name="c", subcore_axis_name="s")

    @functools.partial(
        pl.kernel, mesh=mesh,
        compiler_params=pltpu.CompilerParams(needs_layout_passes=False),
        out_type=jax.ShapeDtypeStruct((_BATCH, _PAD), jnp.float32),
        scratch_types=[
            pltpu.VMEM((b_per_w,), jnp.int32),
            pltpu.VMEM((_LATENT_DIM, _NUM_EMBED), jnp.float32),
            pltpu.VMEM_SHARED((_LATENT_DIM, _NUM_EMBED), jnp.float32),
            pltpu.VMEM((2, qrows, _PAD), jnp.float32),
            pltpu.SemaphoreType.DMA,
        ],
    )
    def gather(e_hbm, idx_hbm, out_hbm, idx_v, e_v, e_sh, bufs, ssem):
        sid = lax.axis_index("s")
        wid = sid * nc + lax.axis_index("c")
        base = wid * b_per_w
        rows_per_tile = _LATENT_DIM // ns
        pltpu.sync_copy(idx_hbm.at[pl.ds(base, b_per_w)], idx_v)
        pltpu.sync_copy(
            e_hbm.at[pl.ds(sid * rows_per_tile, rows_per_tile)],
            e_sh.at[pl.ds(sid * rows_per_tile, rows_per_tile)],
        )
        plsc.subcore_barrier()
        pltpu.sync_copy(e_sh, e_v)
        lanes = jax.lax.iota(jnp.int32, 16)
        h_s = [None] * 4
        for qh in range(4):
            if qh >= 2:
                h_s[qh - 2].wait()
            bvec = jnp.full((16,), qh % 2, jnp.int32)
            for g in range(n_grp):
                row0 = g * 16
                gidx = plsc.load_gather(idx_v, [row0 + qh * qrows + lanes])
                for c in range(_LATENT_DIM):
                    cvec = jnp.full((16,), c, jnp.int32)
                    vals = plsc.load_gather(e_v, [cvec, gidx])
                    plsc.store_scatter(bufs, [bvec, row0 + lanes, cvec], vals)
            h_s[qh] = pltpu.async_copy(
                bufs.at[qh % 2],
                out_hbm.at[pl.ds(base + qh * qrows, qrows)],
                ssem,
            )
        h_s[2].wait()
        h_s[3].wait()

    return gather


def _decode_body(q_ref, wd_ref, bd_ref, out_ref):
    q = q_ref[...][:, :_LATENT_DIM]
    out_ref[...] = (
        jnp.dot(q, wd_ref[...], preferred_element_type=jnp.float32)
        + bd_ref[...]
    )


def _decode(quantized, W_dec, b_dec):
    return pl.pallas_call(
        _decode_body,
        grid=(_BATCH // _BM_DEC,),
        in_specs=[
            pl.BlockSpec((_BM_DEC, _PAD), lambda i: (i, 0)),
            pl.BlockSpec((_LATENT_DIM, _INPUT_DIM), lambda i: (0, 0)),
            pl.BlockSpec((1, _INPUT_DIM), lambda i: (0, 0)),
        ],
        out_specs=pl.BlockSpec((_BM_DEC, _INPUT_DIM), lambda i: (i, 0)),
        out_shape=jax.ShapeDtypeStruct((_BATCH, _INPUT_DIM), jnp.float32),
    )(quantized, W_dec, b_dec.reshape(1, _INPUT_DIM))


def kernel(x, W_enc, b_enc, W_dec, b_dec, embeddings):
    idx = _compute_indices(x, W_enc, b_enc, embeddings)
    quantized = _make_gather()(embeddings, idx)
    return _decode(quantized, W_dec, b_dec)
